# baseline (speedup 1.0000x reference)
import jax
import jax.numpy as jnp
from jax.experimental import pallas as pl
from jax.experimental.pallas import tpu as pltpu


def _copy_body(xu_ref, xi_ref, ou_ref, oi_ref):
    ou_ref[...] = xu_ref[...]
    oi_ref[...] = xi_ref[...]


def kernel(x_user, x_item, edge_index):
    del edge_index
    n, d = x_user.shape
    blk = 10000
    grid = (n // blk,)
    spec = pl.BlockSpec((blk, d), lambda i: (i, 0))
    ou, oi = pl.pallas_call(
        _copy_body,
        grid=grid,
        in_specs=[spec, spec],
        out_specs=[spec, spec],
        out_shape=[
            jax.ShapeDtypeStruct((n, d), x_user.dtype),
            jax.ShapeDtypeStruct((n, d), x_item.dtype),
        ],
    )(x_user, x_item)
    return (ou, oi)


# speedup vs baseline: 1.0772x; 1.0772x over previous
import jax
import jax.numpy as jnp
from jax.experimental import pallas as pl
from jax.experimental.pallas import tpu as pltpu

_CHUNKS = 5
_ROWS = 2000


def _body(xu, xi, ou, oi, vu, vi, sin_u, sin_i, sout_u, sout_i):
    for k in range(_CHUNKS):
        sl = pl.ds(k * _ROWS, _ROWS)
        pltpu.make_async_copy(xu.at[sl], vu.at[sl], sin_u.at[k]).start()
        pltpu.make_async_copy(xi.at[sl], vi.at[sl], sin_i.at[k]).start()
    for k in range(_CHUNKS):
        sl = pl.ds(k * _ROWS, _ROWS)
        pltpu.make_async_copy(xu.at[sl], vu.at[sl], sin_u.at[k]).wait()
        pltpu.make_async_copy(vu.at[sl], ou.at[sl], sout_u.at[k]).start()
        pltpu.make_async_copy(xi.at[sl], vi.at[sl], sin_i.at[k]).wait()
        pltpu.make_async_copy(vi.at[sl], oi.at[sl], sout_i.at[k]).start()
    for k in range(_CHUNKS):
        sl = pl.ds(k * _ROWS, _ROWS)
        pltpu.make_async_copy(vu.at[sl], ou.at[sl], sout_u.at[k]).wait()
        pltpu.make_async_copy(vi.at[sl], oi.at[sl], sout_i.at[k]).wait()


def kernel(x_user, x_item, edge_index):
    del edge_index
    n, d = x_user.shape
    ou, oi = pl.pallas_call(
        _body,
        in_specs=[
            pl.BlockSpec(memory_space=pl.ANY),
            pl.BlockSpec(memory_space=pl.ANY),
        ],
        out_specs=[
            pl.BlockSpec(memory_space=pl.ANY),
            pl.BlockSpec(memory_space=pl.ANY),
        ],
        out_shape=[
            jax.ShapeDtypeStruct((n, d), x_user.dtype),
            jax.ShapeDtypeStruct((n, d), x_item.dtype),
        ],
        scratch_shapes=[
            pltpu.VMEM((n, d), jnp.float32),
            pltpu.VMEM((n, d), jnp.float32),
            pltpu.SemaphoreType.DMA((_CHUNKS,)),
            pltpu.SemaphoreType.DMA((_CHUNKS,)),
            pltpu.SemaphoreType.DMA((_CHUNKS,)),
            pltpu.SemaphoreType.DMA((_CHUNKS,)),
        ],
    )(x_user, x_item)
    return (ou, oi)


# speedup vs baseline: 1.1146x; 1.0347x over previous
import jax
import jax.numpy as jnp
from jax.experimental import pallas as pl
from jax.experimental.pallas import tpu as pltpu

_CHUNKS = 2
_ROWS = 5000


def _body(xu, xi, ou, oi, vu, vi, sin_u, sin_i, sout_u, sout_i):
    for k in range(_CHUNKS):
        sl = pl.ds(k * _ROWS, _ROWS)
        pltpu.make_async_copy(xu.at[sl], vu.at[sl], sin_u.at[k]).start()
        pltpu.make_async_copy(xi.at[sl], vi.at[sl], sin_i.at[k]).start()
    for k in range(_CHUNKS):
        sl = pl.ds(k * _ROWS, _ROWS)
        pltpu.make_async_copy(xu.at[sl], vu.at[sl], sin_u.at[k]).wait()
        pltpu.make_async_copy(vu.at[sl], ou.at[sl], sout_u.at[k]).start()
        pltpu.make_async_copy(xi.at[sl], vi.at[sl], sin_i.at[k]).wait()
        pltpu.make_async_copy(vi.at[sl], oi.at[sl], sout_i.at[k]).start()
    for k in range(_CHUNKS):
        sl = pl.ds(k * _ROWS, _ROWS)
        pltpu.make_async_copy(vu.at[sl], ou.at[sl], sout_u.at[k]).wait()
        pltpu.make_async_copy(vi.at[sl], oi.at[sl], sout_i.at[k]).wait()


def kernel(x_user, x_item, edge_index):
    del edge_index
    n, d = x_user.shape
    ou, oi = pl.pallas_call(
        _body,
        in_specs=[
            pl.BlockSpec(memory_space=pl.ANY),
            pl.BlockSpec(memory_space=pl.ANY),
        ],
        out_specs=[
            pl.BlockSpec(memory_space=pl.ANY),
            pl.BlockSpec(memory_space=pl.ANY),
        ],
        out_shape=[
            jax.ShapeDtypeStruct((n, d), x_user.dtype),
            jax.ShapeDtypeStruct((n, d), x_item.dtype),
        ],
        scratch_shapes=[
            pltpu.VMEM((n, d), jnp.float32),
            pltpu.VMEM((n, d), jnp.float32),
            pltpu.SemaphoreType.DMA((_CHUNKS,)),
            pltpu.SemaphoreType.DMA((_CHUNKS,)),
            pltpu.SemaphoreType.DMA((_CHUNKS,)),
            pltpu.SemaphoreType.DMA((_CHUNKS,)),
        ],
    )(x_user, x_item)
    return (ou, oi)
